# baseline (speedup 1.0000x reference)
import functools

import jax
import jax.numpy as jnp
from jax import lax
from jax.experimental import pallas as pl
from jax.experimental.pallas import tpu as pltpu
from jax.experimental.pallas import tpu_sc as plsc

_ROWS = 128
_COLS = 32768
_L = 16
_NW = 32
_RPW = _ROWS // _NW


def _sc_body(x_hbm, o_hbm, buf, sem):
    wid = lax.axis_index("s") * 2 + lax.axis_index("c")

    def do_row(r, _):
        row = wid * _RPW + r
        pltpu.async_copy(x_hbm.at[row], buf, sem).wait()

        def vreg_step(i, carry):
            v = buf[pl.ds(i * _L, _L)]
            s = plsc.cumsum(v)
            buf[pl.ds(i * _L, _L)] = s + carry
            return carry + jnp.sum(v)

        lax.fori_loop(0, _COLS // _L, vreg_step, jnp.float32(0.0),
                      unroll=8)
        pltpu.async_copy(buf, o_hbm.at[row], sem).wait()
        return _

    lax.fori_loop(0, _RPW, do_row, 0)


def kernel(x):
    mesh = plsc.VectorSubcoreMesh(core_axis_name="c", subcore_axis_name="s")
    f = pl.kernel(
        _sc_body,
        out_type=jax.ShapeDtypeStruct((_ROWS, _COLS), jnp.float32),
        mesh=mesh,
        scratch_types=[
            pltpu.VMEM((_COLS,), jnp.float32),
            pltpu.SemaphoreType.DMA,
        ],
        compiler_params=pltpu.CompilerParams(needs_layout_passes=False),
    )
    return f(x)
